# baseline (speedup 1.0000x reference)
import functools

import jax
import jax.numpy as jnp
from jax import lax
from jax.experimental import pallas as pl
from jax.experimental.pallas import tpu as pltpu
from jax.experimental.pallas import tpu_sc as plsc

BATCH = 16384
NUM_ENT = 1000000
NUM_REL = 1000
EMB_DIM = 64
PAIR = 2 * EMB_DIM
LANES = 16
NUM_CORES = 2
NUM_SUBCORES = 16
NUM_WORKERS = NUM_CORES * NUM_SUBCORES
ROWS_PER_WORKER = BATCH // NUM_WORKERS
CHUNK = 128
NUM_CHUNKS = ROWS_PER_WORKER // CHUNK
BLOCKS_PER_CHUNK = CHUNK // LANES
KCH = EMB_DIM // LANES

TCOLS = NUM_ENT // 128
TCOLS_BASE = TCOLS // NUM_WORKERS
TCOLS_EXTRA = TCOLS % NUM_WORKERS
PIECE_E = 512
PIECE_P = PIECE_E // 2
NPIECES = (TCOLS_BASE * 128) // PIECE_E
TAIL_E = 128
REM_E = NUM_ENT - TCOLS * 128
NUM_PAIR = NUM_ENT // 2

_mesh = plsc.VectorSubcoreMesh(core_axis_name="c", subcore_axis_name="s")

_params = pltpu.CompilerParams(
    needs_layout_passes=False, use_tc_tiling_on_sc=True)


@functools.partial(
    pl.kernel,
    mesh=_mesh,
    compiler_params=_params,
    out_type=jax.ShapeDtypeStruct((NUM_PAIR, PAIR), jnp.float32),
    scratch_types=[
        pltpu.VMEM((2, EMB_DIM, PIECE_E), jnp.float32),
        pltpu.VMEM((EMB_DIM, TAIL_E), jnp.float32),
        pltpu.VMEM((EMB_DIM, REM_E), jnp.float32),
        pltpu.VMEM((PIECE_P, PAIR), jnp.float32),
        pltpu.SemaphoreType.DMA,
    ],
)
def _pairify_sc(ent_hbm, pair_hbm, in_v, tail_v, rem_v, o_v, sem):
    wid = lax.axis_index("s") * NUM_CORES + lax.axis_index("c")
    ebase = pl.multiple_of(
        (TCOLS_BASE * wid + lax.min(wid, TCOLS_EXTRA)) * 128, 128)
    pbase = pl.multiple_of(ebase // 2, 64)
    iota = lax.iota(jnp.int32, LANES)

    def issue(jp, buf):
        return pltpu.async_copy(
            ent_hbm.at[:, pl.ds(pl.multiple_of(ebase + jp * PIECE_E, 128),
                                PIECE_E)],
            in_v.at[buf], sem)

    issue(0, 0)

    def piece_body(jp, carry):
        buf = lax.rem(jp, 2)

        @pl.when(jp < NPIECES - 1)
        def _():
            issue(jp + 1, 1 - buf)

        pltpu.make_async_copy(
            ent_hbm.at[:, pl.ds(0, PIECE_E)], in_v.at[buf], sem).wait()

        def row_body(p, rcarry):
            for h in range(2):
                e_loc = jnp.full((LANES,), 2 * p + h, jnp.int32)
                for k in range(KCH):
                    v = plsc.load_gather(in_v.at[buf], [k * LANES + iota, e_loc])
                    o_v[p, pl.ds(h * EMB_DIM + k * LANES, LANES)] = v
            return rcarry

        lax.fori_loop(0, PIECE_P, row_body, 0)
        pltpu.sync_copy(
            o_v,
            pair_hbm.at[pl.ds(pl.multiple_of(pbase + jp * PIECE_P, 64),
                              PIECE_P)])
        return carry

    lax.fori_loop(0, NPIECES, piece_body, 0)

    @pl.when(wid < TCOLS_EXTRA)
    def _tail():
        pltpu.async_copy(
            ent_hbm.at[:, pl.ds(pl.multiple_of(ebase + NPIECES * PIECE_E, 128),
                                TAIL_E)],
            tail_v, sem).wait()

        def row_body(p, rcarry):
            for h in range(2):
                e_loc = jnp.full((LANES,), 2 * p + h, jnp.int32)
                for k in range(KCH):
                    v = plsc.load_gather(tail_v, [k * LANES + iota, e_loc])
                    o_v[p, pl.ds(h * EMB_DIM + k * LANES, LANES)] = v
            return rcarry

        lax.fori_loop(0, TAIL_E // 2, row_body, 0)
        pltpu.sync_copy(
            o_v.at[pl.ds(0, TAIL_E // 2)],
            pair_hbm.at[pl.ds(
                pl.multiple_of(pbase + NPIECES * PIECE_P, 64), TAIL_E // 2)])

    @pl.when(wid == NUM_WORKERS - 1)
    def _rem():
        pltpu.async_copy(
            ent_hbm.at[:, pl.ds(TCOLS * 128, REM_E)], rem_v, sem).wait()

        def row_body(p, rcarry):
            for h in range(2):
                e_loc = jnp.full((LANES,), 2 * p + h, jnp.int32)
                for k in range(KCH):
                    v = plsc.load_gather(rem_v, [k * LANES + iota, e_loc])
                    o_v[p, pl.ds(h * EMB_DIM + k * LANES, LANES)] = v
            return rcarry

        lax.fori_loop(0, REM_E // 2, row_body, 0)
        pltpu.sync_copy(
            o_v.at[pl.ds(0, REM_E // 2)],
            pair_hbm.at[pl.ds(NUM_PAIR - REM_E // 2, REM_E // 2)])


@functools.partial(
    pl.kernel,
    mesh=_mesh,
    compiler_params=_params,
    out_type=jax.ShapeDtypeStruct((BATCH,), jnp.float32),
    scratch_types=[
        pltpu.VMEM((ROWS_PER_WORKER,), jnp.int32),
        pltpu.VMEM((ROWS_PER_WORKER,), jnp.int32),
        pltpu.VMEM((ROWS_PER_WORKER,), jnp.int32),
        pltpu.VMEM((3 * 2, CHUNK), jnp.int32),
        pltpu.VMEM((2, CHUNK, PAIR), jnp.float32),
        pltpu.VMEM((2, CHUNK, PAIR), jnp.float32),
        pltpu.VMEM((2, CHUNK, PAIR), jnp.float32),
        pltpu.VMEM((LANES * LANES,), jnp.float32),
        pltpu.VMEM((ROWS_PER_WORKER,), jnp.float32),
        pltpu.SemaphoreType.DMA,
        pltpu.SemaphoreType.DMA,
    ],
)
def _distmult_sc(hs_hbm, rs_hbm, ts_hbm, ent_hbm, rel_hbm, out_hbm,
                 hs_v, rs_v, ts_v, half_v, eh_v, er_v, et_v, blk_v, o_v,
                 sem_i, sem_g):
    wid = lax.axis_index("s") * NUM_CORES + lax.axis_index("c")
    base = wid * ROWS_PER_WORKER

    idx_copies = [
        pltpu.async_copy(hs_hbm.at[pl.ds(base, ROWS_PER_WORKER)], hs_v, sem_i),
        pltpu.async_copy(rs_hbm.at[pl.ds(base, ROWS_PER_WORKER)], rs_v, sem_i),
        pltpu.async_copy(ts_hbm.at[pl.ds(base, ROWS_PER_WORKER)], ts_v, sem_i),
    ]
    for c in idx_copies:
        c.wait()

    def build_half_idx(jc, src_v, slot):
        for g in range(CHUNK // LANES):
            vec = src_v[pl.ds(jc * CHUNK + g * LANES, LANES)]
            half_v[slot, pl.ds(g * LANES, LANES)] = lax.shift_right_logical(
                vec, 1)

    def issue_chunk(jc):
        buf = jc % 2
        build_half_idx(jc, hs_v, buf * 3 + 0)
        build_half_idx(jc, rs_v, buf * 3 + 1)
        build_half_idx(jc, ts_v, buf * 3 + 2)
        return [
            pltpu.async_copy(
                ent_hbm.at[half_v.at[buf * 3 + 0]], eh_v.at[buf], sem_g),
            pltpu.async_copy(
                rel_hbm.at[half_v.at[buf * 3 + 1]], er_v.at[buf], sem_g),
            pltpu.async_copy(
                ent_hbm.at[half_v.at[buf * 3 + 2]], et_v.at[buf], sem_g),
        ]

    pending = issue_chunk(0)
    for jc in range(NUM_CHUNKS):
        buf = jc % 2
        nxt = issue_chunk(jc + 1) if jc + 1 < NUM_CHUNKS else []
        for c in pending:
            c.wait()
        pending = nxt

        def blk_body(blk, bcarry):
            row0 = blk * LANES
            hvec = hs_v[pl.ds(jc * CHUNK + row0, LANES)]
            rvec = rs_v[pl.ds(jc * CHUNK + row0, LANES)]
            tvec = ts_v[pl.ds(jc * CHUNK + row0, LANES)]
            hpar = (hvec & 1) * EMB_DIM
            rpar = (rvec & 1) * EMB_DIM
            tpar = (tvec & 1) * EMB_DIM
            for i in range(LANES):
                r = row0 + i
                ho = hpar[i]
                ro = rpar[i]
                to = tpar[i]
                acc = None
                for k in range(KCH):
                    prod = (eh_v[buf, r, pl.ds(ho + k * LANES, LANES)]
                            * er_v[buf, r, pl.ds(ro + k * LANES, LANES)]
                            ) * et_v[buf, r, pl.ds(to + k * LANES, LANES)]
                    acc = prod if acc is None else acc + prod
                blk_v[pl.ds(i * LANES, LANES)] = acc
            t_idx = lax.iota(jnp.int32, LANES) * LANES
            res = plsc.load_gather(blk_v, [t_idx])
            for i in range(1, LANES):
                res = res + plsc.load_gather(blk_v, [t_idx + i])
            o_v[pl.ds(jc * CHUNK + row0, LANES)] = res
            return bcarry

        lax.fori_loop(0, BLOCKS_PER_CHUNK, blk_body, 0)

    pltpu.sync_copy(o_v, out_hbm.at[pl.ds(base, ROWS_PER_WORKER)])


def kernel(hs, rs, ts, ent_embs, rel_embs):
    pair = _pairify_sc(ent_embs.T)
    rel2 = jnp.reshape(rel_embs, (NUM_REL // 2, PAIR))
    return _distmult_sc(hs, rs, ts, pair, rel2)


# speedup vs baseline: 2.5603x; 2.5603x over previous
import functools

import jax
import jax.numpy as jnp
from jax import lax
from jax.experimental import pallas as pl
from jax.experimental.pallas import tpu as pltpu
from jax.experimental.pallas import tpu_sc as plsc

BATCH = 16384
NUM_ENT = 1000000
NUM_REL = 1000
EMB_DIM = 64
PAIR = 2 * EMB_DIM
LANES = 16
NUM_CORES = 2
NUM_SUBCORES = 16
NUM_WORKERS = NUM_CORES * NUM_SUBCORES
ROWS_PER_WORKER = BATCH // NUM_WORKERS
CHUNK = 128
NUM_CHUNKS = ROWS_PER_WORKER // CHUNK
BLOCKS_PER_CHUNK = CHUNK // LANES
KCH = EMB_DIM // LANES

_mesh = plsc.VectorSubcoreMesh(core_axis_name="c", subcore_axis_name="s")


@functools.partial(
    pl.kernel,
    mesh=_mesh,
    compiler_params=pltpu.CompilerParams(
        needs_layout_passes=False, use_tc_tiling_on_sc=True,
        skip_device_barrier=True, disable_bounds_checks=True),
    out_type=jax.ShapeDtypeStruct((BATCH,), jnp.float32),
    scratch_types=[
        pltpu.VMEM((ROWS_PER_WORKER,), jnp.int32),
        pltpu.VMEM((ROWS_PER_WORKER,), jnp.int32),
        pltpu.VMEM((ROWS_PER_WORKER,), jnp.int32),
        pltpu.VMEM((3 * 2, CHUNK), jnp.int32),
        pltpu.VMEM((2, CHUNK, PAIR), jnp.float32),
        pltpu.VMEM((2, CHUNK, PAIR), jnp.float32),
        pltpu.VMEM((2, CHUNK, PAIR), jnp.float32),
        pltpu.VMEM((LANES * LANES,), jnp.float32),
        pltpu.VMEM((ROWS_PER_WORKER,), jnp.float32),
        pltpu.SemaphoreType.DMA,
        pltpu.SemaphoreType.DMA,
    ],
)
def _distmult_sc(hs_hbm, rs_hbm, ts_hbm, ent_hbm, rel_hbm, out_hbm,
                 hs_v, rs_v, ts_v, half_v, eh_v, er_v, et_v, blk_v, o_v,
                 sem_i, sem_g):
    wid = lax.axis_index("s") * NUM_CORES + lax.axis_index("c")
    base = wid * ROWS_PER_WORKER

    idx_copies = [
        pltpu.async_copy(hs_hbm.at[pl.ds(base, ROWS_PER_WORKER)], hs_v, sem_i),
        pltpu.async_copy(rs_hbm.at[pl.ds(base, ROWS_PER_WORKER)], rs_v, sem_i),
        pltpu.async_copy(ts_hbm.at[pl.ds(base, ROWS_PER_WORKER)], ts_v, sem_i),
    ]
    for c in idx_copies:
        c.wait()

    def build_half_idx(jc, src_v, slot):
        for g in range(CHUNK // LANES):
            vec = src_v[pl.ds(jc * CHUNK + g * LANES, LANES)]
            half_v[slot, pl.ds(g * LANES, LANES)] = lax.shift_right_logical(
                vec, 1)

    def issue_chunk(jc):
        buf = jc % 2
        build_half_idx(jc, hs_v, buf * 3 + 0)
        build_half_idx(jc, rs_v, buf * 3 + 1)
        build_half_idx(jc, ts_v, buf * 3 + 2)
        return [
            pltpu.async_copy(
                ent_hbm.at[half_v.at[buf * 3 + 0]], eh_v.at[buf], sem_g),
            pltpu.async_copy(
                rel_hbm.at[half_v.at[buf * 3 + 1]], er_v.at[buf], sem_g),
            pltpu.async_copy(
                ent_hbm.at[half_v.at[buf * 3 + 2]], et_v.at[buf], sem_g),
        ]

    pending = issue_chunk(0)
    for jc in range(NUM_CHUNKS):
        buf = jc % 2
        nxt = issue_chunk(jc + 1) if jc + 1 < NUM_CHUNKS else []
        for c in pending:
            c.wait()
        pending = nxt

        def blk_body(blk, bcarry):
            row0 = blk * LANES
            hvec = hs_v[pl.ds(jc * CHUNK + row0, LANES)]
            rvec = rs_v[pl.ds(jc * CHUNK + row0, LANES)]
            tvec = ts_v[pl.ds(jc * CHUNK + row0, LANES)]
            hpar = (hvec & 1) * EMB_DIM
            rpar = (rvec & 1) * EMB_DIM
            tpar = (tvec & 1) * EMB_DIM
            for i in range(LANES):
                r = row0 + i
                ho = hpar[i]
                ro = rpar[i]
                to = tpar[i]
                acc = None
                for k in range(KCH):
                    prod = (eh_v[buf, r, pl.ds(ho + k * LANES, LANES)]
                            * er_v[buf, r, pl.ds(ro + k * LANES, LANES)]
                            ) * et_v[buf, r, pl.ds(to + k * LANES, LANES)]
                    acc = prod if acc is None else acc + prod
                blk_v[pl.ds(i * LANES, LANES)] = acc
            t_idx = lax.iota(jnp.int32, LANES) * LANES
            res = plsc.load_gather(blk_v, [t_idx])
            for i in range(1, LANES):
                res = res + plsc.load_gather(blk_v, [t_idx + i])
            o_v[pl.ds(jc * CHUNK + row0, LANES)] = res
            return bcarry

        lax.fori_loop(0, BLOCKS_PER_CHUNK, blk_body, 0)

    pltpu.sync_copy(o_v, out_hbm.at[pl.ds(base, ROWS_PER_WORKER)])


def kernel(hs, rs, ts, ent_embs, rel_embs):
    ent2 = jnp.reshape(ent_embs, (NUM_ENT // 2, PAIR))
    rel2 = jnp.reshape(rel_embs, (NUM_REL // 2, PAIR))
    return _distmult_sc(hs, rs, ts, ent2, rel2)
